# baseline (speedup 1.0000x reference)
import functools

import jax
import jax.numpy as jnp
from jax import lax
from jax.experimental import pallas as pl
from jax.experimental.pallas import tpu as pltpu
from jax.experimental.pallas import tpu_sc as plsc

NUM_CORES = 2
NUM_SUBCORES = 16
LANES = 16
NW = NUM_CORES * NUM_SUBCORES

BATCH = 16384
EMBED_DIM = 64
B_PER_W = BATCH // NW
IDX_CHUNK = 128
N_CHUNKS = B_PER_W // IDX_CHUNK


def _body(user_ids_hbm, item_ids_hbm, user_table_hbm, item_table_hbm, out_hbm,
          uidx_v, iidx_v, urows_v, irows_v, out_v, sem):
    wid = lax.axis_index("s") * NUM_CORES + lax.axis_index("c")
    base = wid * B_PER_W

    for j in range(N_CHUNKS):
        pltpu.sync_copy(user_ids_hbm.at[pl.ds(base + j * IDX_CHUNK, IDX_CHUNK)],
                        uidx_v.at[j])
        pltpu.sync_copy(item_ids_hbm.at[pl.ds(base + j * IDX_CHUNK, IDX_CHUNK)],
                        iidx_v.at[j])

    copies = []
    for j in range(N_CHUNKS):
        copies.append(pltpu.async_copy(
            user_table_hbm.at[uidx_v.at[j]],
            urows_v.at[pl.ds(j * IDX_CHUNK, IDX_CHUNK)], sem))
        copies.append(pltpu.async_copy(
            item_table_hbm.at[iidx_v.at[j]],
            irows_v.at[pl.ds(j * IDX_CHUNK, IDX_CHUNK)], sem))
    for c in copies:
        c.wait()

    lane = lax.iota(jnp.int32, LANES)

    def group_body(g, _):
        b0 = g * LANES
        out16 = jnp.zeros((LANES,), jnp.float32)
        for l in range(LANES):
            b = b0 + l
            acc = urows_v[b, pl.ds(0, LANES)] * irows_v[b, pl.ds(0, LANES)]
            for c in range(1, EMBED_DIM // LANES):
                acc += (urows_v[b, pl.ds(c * LANES, LANES)]
                        * irows_v[b, pl.ds(c * LANES, LANES)])
            out16 = jnp.where(lane == l, jnp.sum(acc), out16)
        out_v[pl.ds(b0, LANES)] = out16
        return _

    lax.fori_loop(0, B_PER_W // LANES, group_body, 0)

    pltpu.sync_copy(out_v, out_hbm.at[pl.ds(base, B_PER_W)])


@jax.jit
def _bprmf_score(user_ids, item_ids, user_table, item_table):
    mesh = plsc.VectorSubcoreMesh(core_axis_name="c", subcore_axis_name="s",
                                  num_cores=NUM_CORES,
                                  num_subcores=NUM_SUBCORES)
    return pl.kernel(
        _body,
        out_type=jax.ShapeDtypeStruct((BATCH,), jnp.float32),
        mesh=mesh,
        compiler_params=pltpu.CompilerParams(needs_layout_passes=False,
                                             use_tc_tiling_on_sc=False),
        scratch_types=[
            pltpu.VMEM((N_CHUNKS, IDX_CHUNK), jnp.int32),
            pltpu.VMEM((N_CHUNKS, IDX_CHUNK), jnp.int32),
            pltpu.VMEM((B_PER_W, EMBED_DIM), jnp.float32),
            pltpu.VMEM((B_PER_W, EMBED_DIM), jnp.float32),
            pltpu.VMEM((B_PER_W,), jnp.float32),
            pltpu.SemaphoreType.DMA,
        ],
    )(user_ids, item_ids, user_table, item_table)


def kernel(user_ids, item_ids, user_table, item_table):
    return _bprmf_score(user_ids.astype(jnp.int32), item_ids.astype(jnp.int32),
                        user_table, item_table)


# speedup vs baseline: 2.7217x; 2.7217x over previous
import jax
import jax.numpy as jnp
from jax import lax
from jax.experimental import pallas as pl
from jax.experimental.pallas import tpu as pltpu
from jax.experimental.pallas import tpu_sc as plsc

NUM_CORES = 2
NUM_SUBCORES = 16
LANES = 16
NW = NUM_CORES * NUM_SUBCORES

BATCH = 16384
EMBED_DIM = 64
B_PER_W = BATCH // NW
WIN = 128
NBUF = 4


def _window_copy(tab_hbm, bufs, slot, col, sem):
    start = pl.multiple_of((col >> 7) << 7, WIN)
    return pltpu.async_copy(tab_hbm.at[:, pl.ds(start, WIN)], bufs.at[slot],
                            sem)


def _body(user_ids_hbm, item_ids_hbm, ut_hbm, it_hbm, out_hbm,
          uidx_v, iidx_v, ubufs, ibufs, out_v, sem_u, sem_i):
    wid = lax.axis_index("s") * NUM_CORES + lax.axis_index("c")
    base = wid * B_PER_W

    pltpu.sync_copy(user_ids_hbm.at[pl.ds(base, B_PER_W)],
                    uidx_v.at[pl.ds(0, B_PER_W)])
    pltpu.sync_copy(item_ids_hbm.at[pl.ds(base, B_PER_W)],
                    iidx_v.at[pl.ds(0, B_PER_W)])

    lane = lax.iota(jnp.int32, LANES)

    def ids_at(b):
        return uidx_v[pl.ds(b, LANES)][0], iidx_v[pl.ds(b, LANES)][0]

    for b in range(NBUF - 1):
        uid, iid = ids_at(b)
        _window_copy(ut_hbm, ubufs, b, uid, sem_u)
        _window_copy(it_hbm, ibufs, b, iid, sem_i)

    def b_body(b, out16):
        p = b & (NBUF - 1)
        pltpu.make_async_copy(ut_hbm.at[:, pl.ds(0, WIN)], ubufs.at[p],
                              sem_u).wait()
        pltpu.make_async_copy(it_hbm.at[:, pl.ds(0, WIN)], ibufs.at[p],
                              sem_i).wait()

        bn = b + NBUF - 1

        @pl.when(bn < B_PER_W)
        def _():
            uid_n, iid_n = ids_at(bn)
            pn = bn & (NBUF - 1)
            _window_copy(ut_hbm, ubufs, pn, uid_n, sem_u)
            _window_copy(it_hbm, ibufs, pn, iid_n, sem_i)

        uid, iid = ids_at(b)
        uoff = jnp.full((LANES,), uid & (WIN - 1), jnp.int32)
        ioff = jnp.full((LANES,), iid & (WIN - 1), jnp.int32)
        pv = jnp.full((LANES,), p, jnp.int32)
        acc = jnp.zeros((LANES,), jnp.float32)
        for jc in range(EMBED_DIM // LANES):
            jrow = jc * LANES + lane
            u = plsc.load_gather(ubufs, [pv, jrow, uoff])
            iv = plsc.load_gather(ibufs, [pv, jrow, ioff])
            acc += u * iv

        l = b & (LANES - 1)
        out16 = jnp.where(l == 0, jnp.zeros((LANES,), jnp.float32), out16)
        out16 = jnp.where(lane == l, jnp.sum(acc), out16)

        @pl.when(l == LANES - 1)
        def _():
            out_v[pl.ds(b - (LANES - 1), LANES)] = out16

        return out16

    lax.fori_loop(0, B_PER_W, b_body, jnp.zeros((LANES,), jnp.float32))

    pltpu.sync_copy(out_v, out_hbm.at[pl.ds(base, B_PER_W)])


@jax.jit
def _bprmf_score(user_ids, item_ids, user_table, item_table):
    mesh = plsc.VectorSubcoreMesh(core_axis_name="c", subcore_axis_name="s",
                                  num_cores=NUM_CORES,
                                  num_subcores=NUM_SUBCORES)
    return pl.kernel(
        _body,
        out_type=jax.ShapeDtypeStruct((BATCH,), jnp.float32),
        mesh=mesh,
        compiler_params=pltpu.CompilerParams(needs_layout_passes=False),
        scratch_types=[
            pltpu.VMEM((B_PER_W + LANES,), jnp.int32),
            pltpu.VMEM((B_PER_W + LANES,), jnp.int32),
            pltpu.VMEM((NBUF, EMBED_DIM, WIN), jnp.float32),
            pltpu.VMEM((NBUF, EMBED_DIM, WIN), jnp.float32),
            pltpu.VMEM((B_PER_W,), jnp.float32),
            pltpu.SemaphoreType.DMA,
            pltpu.SemaphoreType.DMA,
        ],
    )(user_ids, item_ids, user_table.T, item_table.T)


def kernel(user_ids, item_ids, user_table, item_table):
    return _bprmf_score(user_ids.astype(jnp.int32), item_ids.astype(jnp.int32),
                        user_table, item_table)


# speedup vs baseline: 2.9500x; 1.0839x over previous
import jax
import jax.numpy as jnp
from jax import lax
from jax.experimental import pallas as pl
from jax.experimental.pallas import tpu as pltpu
from jax.experimental.pallas import tpu_sc as plsc

NUM_CORES = 2
NUM_SUBCORES = 16
LANES = 16
NW = NUM_CORES * NUM_SUBCORES

BATCH = 16384
EMBED_DIM = 64
B_PER_W = BATCH // NW
WIN = 128
NBUF = 6


def _window_copy(tab_hbm, bufs, slot, col, sem):
    start = pl.multiple_of((col >> 7) << 7, WIN)
    return pltpu.async_copy(tab_hbm.at[:, pl.ds(start, WIN)], bufs.at[slot],
                            sem)


def _body(user_ids_hbm, item_ids_hbm, ut_hbm, it_hbm, out_hbm,
          uidx_v, iidx_v, ubufs, ibufs, out_v, sem_u, sem_i):
    wid = lax.axis_index("s") * NUM_CORES + lax.axis_index("c")
    base = wid * B_PER_W

    pltpu.sync_copy(user_ids_hbm.at[pl.ds(base, B_PER_W)],
                    uidx_v.at[pl.ds(0, B_PER_W)])
    pltpu.sync_copy(item_ids_hbm.at[pl.ds(base, B_PER_W)],
                    iidx_v.at[pl.ds(0, B_PER_W)])

    lane = lax.iota(jnp.int32, LANES)

    def ids_at(b):
        return uidx_v[pl.ds(b, LANES)][0], iidx_v[pl.ds(b, LANES)][0]

    for b in range(NBUF - 1):
        uid, iid = ids_at(b)
        _window_copy(ut_hbm, ubufs, b, uid, sem_u)
        _window_copy(it_hbm, ibufs, b, iid, sem_i)

    def b_body(b, out16):
        p = lax.rem(b, NBUF)
        pltpu.make_async_copy(ut_hbm.at[:, pl.ds(0, WIN)], ubufs.at[p],
                              sem_u).wait()
        pltpu.make_async_copy(it_hbm.at[:, pl.ds(0, WIN)], ibufs.at[p],
                              sem_i).wait()

        bn = b + NBUF - 1

        @pl.when(bn < B_PER_W)
        def _():
            uid_n, iid_n = ids_at(bn)
            pn = lax.rem(bn, NBUF)
            _window_copy(ut_hbm, ubufs, pn, uid_n, sem_u)
            _window_copy(it_hbm, ibufs, pn, iid_n, sem_i)

        uid, iid = ids_at(b)
        uoff = jnp.full((LANES,), uid & (WIN - 1), jnp.int32)
        ioff = jnp.full((LANES,), iid & (WIN - 1), jnp.int32)
        pv = jnp.full((LANES,), p, jnp.int32)
        acc = jnp.zeros((LANES,), jnp.float32)
        for jc in range(EMBED_DIM // LANES):
            jrow = jc * LANES + lane
            u = plsc.load_gather(ubufs, [pv, jrow, uoff])
            iv = plsc.load_gather(ibufs, [pv, jrow, ioff])
            acc += u * iv

        l = b & (LANES - 1)
        out16 = jnp.where(l == 0, jnp.zeros((LANES,), jnp.float32), out16)
        out16 = jnp.where(lane == l, jnp.sum(acc), out16)

        @pl.when(l == LANES - 1)
        def _():
            out_v[pl.ds(b - (LANES - 1), LANES)] = out16

        return out16

    lax.fori_loop(0, B_PER_W, b_body, jnp.zeros((LANES,), jnp.float32))

    pltpu.sync_copy(out_v, out_hbm.at[pl.ds(base, B_PER_W)])


@jax.jit
def _bprmf_score(user_ids, item_ids, user_table, item_table):
    mesh = plsc.VectorSubcoreMesh(core_axis_name="c", subcore_axis_name="s",
                                  num_cores=NUM_CORES,
                                  num_subcores=NUM_SUBCORES)
    return pl.kernel(
        _body,
        out_type=jax.ShapeDtypeStruct((BATCH,), jnp.float32),
        mesh=mesh,
        compiler_params=pltpu.CompilerParams(needs_layout_passes=False),
        scratch_types=[
            pltpu.VMEM((B_PER_W + LANES,), jnp.int32),
            pltpu.VMEM((B_PER_W + LANES,), jnp.int32),
            pltpu.VMEM((NBUF, EMBED_DIM, WIN), jnp.float32),
            pltpu.VMEM((NBUF, EMBED_DIM, WIN), jnp.float32),
            pltpu.VMEM((B_PER_W,), jnp.float32),
            pltpu.SemaphoreType.DMA,
            pltpu.SemaphoreType.DMA,
        ],
    )(user_ids, item_ids, user_table.T, item_table.T)


def kernel(user_ids, item_ids, user_table, item_table):
    return _bprmf_score(user_ids.astype(jnp.int32), item_ids.astype(jnp.int32),
                        user_table, item_table)
